# baseline (speedup 1.0000x reference)
import functools

import jax
import jax.numpy as jnp
from jax import lax
from jax.experimental import pallas as pl
from jax.experimental.pallas import tpu as pltpu
from jax.experimental.pallas import tpu_sc as plsc

_NC = 2
_NS = 16
_NW = _NC * _NS
_R = 32
_L = 16


def _sc_body(B, T, C, x_hbm, pe_hbm, out_hbm, pe_buf, x_buf):
    wid = lax.axis_index("s") * _NC + lax.axis_index("c")
    t_per_w = T // _NW
    n_chunks = t_per_w // _R
    span = _R * C
    n_vregs = span // _L

    def add_loop(i, carry):
        sl = pl.ds(i * _L, _L)
        x_buf[sl] = x_buf[sl] + pe_buf[sl]
        return carry

    def chunk(k, carry):
        t0 = (wid * t_per_w + k * _R) * C
        pltpu.sync_copy(pe_hbm.at[pl.ds(t0, span)], pe_buf)
        for b in range(B):
            pltpu.sync_copy(x_hbm.at[pl.ds(b * T * C + t0, span)], x_buf)
            lax.fori_loop(0, n_vregs, add_loop, 0)
            pltpu.sync_copy(x_buf, out_hbm.at[pl.ds(b * T * C + t0, span)])
        return carry

    lax.fori_loop(0, n_chunks, chunk, 0)


def kernel(x, pos_emb):
    B, T, C = x.shape
    mesh = plsc.VectorSubcoreMesh(core_axis_name="c", subcore_axis_name="s")
    k = pl.kernel(
        functools.partial(_sc_body, B, T, C),
        out_type=jax.ShapeDtypeStruct((B * T * C,), jnp.float32),
        mesh=mesh,
        scratch_types=[
            pltpu.VMEM((_R * C,), jnp.float32),
            pltpu.VMEM((_R * C,), jnp.float32),
        ],
    )
    out = k(x.reshape(-1), pos_emb.reshape(-1))
    return out.reshape(B, T, C)


# speedup vs baseline: 1.4613x; 1.4613x over previous
import functools

import jax
import jax.numpy as jnp
from jax import lax
from jax.experimental import pallas as pl
from jax.experimental.pallas import tpu as pltpu
from jax.experimental.pallas import tpu_sc as plsc

_NC = 2
_NS = 16
_NW = _NC * _NS
_R = 32
_L = 16


def _sc_body(B, T, C, x_hbm, pe_hbm, out_hbm, pe_buf, x_buf):
    wid = lax.axis_index("s") * _NC + lax.axis_index("c")
    t_per_w = T // _NW
    n_chunks = t_per_w // _R
    span = _R * C
    n_vregs = span // _L

    def chunk(k, carry):
        t0 = (wid * t_per_w + k * _R) * C
        pltpu.sync_copy(pe_hbm.at[pl.ds(t0, span)], pe_buf)
        for b in range(B):
            pltpu.sync_copy(x_hbm.at[pl.ds(b * T * C + t0, span)], x_buf)

            @plsc.parallel_loop(0, span, step=_L, unroll=8)
            def add_loop(i):
                sl = pl.ds(i, _L)
                x_buf[sl] = x_buf[sl] + pe_buf[sl]

            pltpu.sync_copy(x_buf, out_hbm.at[pl.ds(b * T * C + t0, span)])
        return carry

    lax.fori_loop(0, n_chunks, chunk, 0)


def kernel(x, pos_emb):
    B, T, C = x.shape
    mesh = plsc.VectorSubcoreMesh(core_axis_name="c", subcore_axis_name="s")
    k = pl.kernel(
        functools.partial(_sc_body, B, T, C),
        out_type=jax.ShapeDtypeStruct((B * T * C,), jnp.float32),
        mesh=mesh,
        scratch_types=[
            pltpu.VMEM((_R * C,), jnp.float32),
            pltpu.VMEM((_R * C,), jnp.float32),
        ],
    )
    out = k(x.reshape(-1), pos_emb.reshape(-1))
    return out.reshape(B, T, C)


# speedup vs baseline: 1.7199x; 1.1769x over previous
import functools

import jax
import jax.numpy as jnp
from jax import lax
from jax.experimental import pallas as pl
from jax.experimental.pallas import tpu as pltpu
from jax.experimental.pallas import tpu_sc as plsc

_NC = 2
_NS = 16
_NW = _NC * _NS
_R = 16
_L = 16


def _sc_body(B, T, C, x_hbm, pe_hbm, out_hbm,
             xb0, xb1, pb0, pb1, ld0, ld1, st0, st1, pe0, pe1):
    wid = lax.axis_index("s") * _NC + lax.axis_index("c")
    t_per_w = T // _NW
    n_chunks = t_per_w // _R
    span = _R * C
    xbuf = [xb0, xb1]
    pbuf = [pb0, pb1]
    ldsem = [ld0, ld1]
    stsem = [st0, st1]
    pesem = [pe0, pe1]
    n_items = n_chunks * B

    def pe_off(k):
        return (wid * t_per_w + k * _R) * C

    def x_off(j):
        k, b = divmod(j, B)
        return b * T * C + pe_off(k)

    pend_ld = [None, None]
    pend_st = [None, None]
    pend_pe = [None, None]

    pend_pe[0] = pltpu.async_copy(pe_hbm.at[pl.ds(pe_off(0), span)],
                                  pbuf[0], pesem[0])
    pend_ld[0] = pltpu.async_copy(x_hbm.at[pl.ds(x_off(0), span)],
                                  xbuf[0], ldsem[0])

    for k in range(n_chunks):
        pend_pe[k % 2].wait()
        for b in range(B):
            j = k * B + b
            cur = j % 2
            pend_ld[cur].wait()
            nxt = (j + 1) % 2
            if j + 1 < n_items:
                if pend_st[nxt] is not None:
                    pend_st[nxt].wait()
                    pend_st[nxt] = None
                pend_ld[nxt] = pltpu.async_copy(
                    x_hbm.at[pl.ds(x_off(j + 1), span)], xbuf[nxt], ldsem[nxt])
            if b == 0 and k + 1 < n_chunks:
                pend_pe[(k + 1) % 2] = pltpu.async_copy(
                    pe_hbm.at[pl.ds(pe_off(k + 1), span)],
                    pbuf[(k + 1) % 2], pesem[(k + 1) % 2])

            xb = xbuf[cur]
            pb = pbuf[k % 2]

            @plsc.parallel_loop(0, span, step=_L, unroll=8)
            def add_loop(i):
                sl = pl.ds(i, _L)
                xb[sl] = xb[sl] + pb[sl]

            pend_st[cur] = pltpu.async_copy(
                xb, out_hbm.at[pl.ds(x_off(j), span)], stsem[cur])

    for h in pend_st:
        if h is not None:
            h.wait()


def kernel(x, pos_emb):
    B, T, C = x.shape
    mesh = plsc.VectorSubcoreMesh(core_axis_name="c", subcore_axis_name="s")
    k = pl.kernel(
        functools.partial(_sc_body, B, T, C),
        out_type=jax.ShapeDtypeStruct((B * T * C,), jnp.float32),
        mesh=mesh,
        scratch_types=[
            pltpu.VMEM((_R * C,), jnp.float32),
            pltpu.VMEM((_R * C,), jnp.float32),
            pltpu.VMEM((_R * C,), jnp.float32),
            pltpu.VMEM((_R * C,), jnp.float32),
            pltpu.SemaphoreType.DMA,
            pltpu.SemaphoreType.DMA,
            pltpu.SemaphoreType.DMA,
            pltpu.SemaphoreType.DMA,
            pltpu.SemaphoreType.DMA,
            pltpu.SemaphoreType.DMA,
        ],
    )
    out = k(x.reshape(-1), pos_emb.reshape(-1))
    return out.reshape(B, T, C)
